# Initial kernel scaffold; baseline (speedup 1.0000x reference)
import jax
import jax.numpy as jnp
from jax.experimental import pallas as pl


def kernel(x, edge_index, W1, W2):
    raise NotImplementedError("write your pallas kernel here")


# speedup vs baseline: 52.0566x; 52.0566x over previous
import functools

import jax
import jax.numpy as jnp
from jax import lax
from jax.experimental import pallas as pl
from jax.experimental.pallas import tpu as pltpu
from jax.experimental.pallas import tpu_sc as plsc

N = 10000
E = 320000
D_IN = 128
D_HID = 16
D_OUT = 128

NC = 2
NS = 16
NW = NC * NS
B = 128
K = 79
EPAD = NW * K * B
NPAD = 10112
RPC = NPAD // NS

_sc_mesh = plsc.VectorSubcoreMesh(
    core_axis_name="c", subcore_axis_name="s", num_cores=NC, num_subcores=NS)


def _deg_body(dsts_hbm, zeros_hbm, ones_hbm, out_hbm,
              acc_sh, dst_kb, ones_v, zbuf, sem):
  c = lax.axis_index("c")
  s = lax.axis_index("s")
  wid = s * NC + c
  row0 = s * RPC
  pltpu.sync_copy(zeros_hbm.at[pl.ds(row0, RPC)], zbuf)
  pltpu.sync_copy(zbuf, acc_sh.at[pl.ds(row0, RPC)])
  pltpu.sync_copy(dsts_hbm.at[wid], dst_kb)
  pltpu.sync_copy(ones_hbm, ones_v)
  plsc.subcore_barrier()

  def step(j, carry):
    pltpu.sync_copy(ones_v, acc_sh.at[dst_kb.at[j]], add=True)
    return carry

  lax.fori_loop(0, K, step, 0)
  plsc.subcore_barrier()
  pltpu.sync_copy(acc_sh.at[pl.ds(row0, RPC)], zbuf)
  pltpu.sync_copy(zbuf, out_hbm.at[c, pl.ds(row0, RPC)])


_sc_params = pltpu.CompilerParams(use_tc_tiling_on_sc=False)

_deg_sc = pl.kernel(
    _deg_body,
    out_type=jax.ShapeDtypeStruct((NC, NPAD, D_HID), jnp.float32),
    mesh=_sc_mesh,
    compiler_params=_sc_params,
    scratch_types=[
        pltpu.VMEM_SHARED((NPAD, D_HID), jnp.float32),
        pltpu.VMEM((K, B), jnp.int32),
        pltpu.VMEM((B, D_HID), jnp.float32),
        pltpu.VMEM((RPC, D_HID), jnp.float32),
        pltpu.SemaphoreType.DMA,
    ],
)


def _agg_body(hs_hbm, srcs_hbm, dsts_hbm, zeros_hbm, out_hbm,
              acc_sh, hs_sh, src_kb, dst_kb, msg_v, zbuf, sem):
  c = lax.axis_index("c")
  s = lax.axis_index("s")
  wid = s * NC + c
  row0 = s * RPC
  rows = pl.ds(row0, RPC)
  pltpu.sync_copy(hs_hbm.at[rows], zbuf)
  pltpu.sync_copy(zbuf, hs_sh.at[rows])
  pltpu.sync_copy(zeros_hbm.at[rows], zbuf)
  pltpu.sync_copy(zbuf, acc_sh.at[rows])
  pltpu.sync_copy(srcs_hbm.at[wid], src_kb)
  pltpu.sync_copy(dsts_hbm.at[wid], dst_kb)
  plsc.subcore_barrier()

  def step(j, carry):
    pltpu.async_copy(hs_sh.at[src_kb.at[j]], msg_v, sem).wait()
    pltpu.sync_copy(msg_v, acc_sh.at[dst_kb.at[j]], add=True)
    return carry

  lax.fori_loop(0, K, step, 0)
  plsc.subcore_barrier()
  pltpu.sync_copy(acc_sh.at[rows], zbuf)
  pltpu.sync_copy(zbuf, out_hbm.at[c, rows])


_agg_sc = pl.kernel(
    _agg_body,
    out_type=jax.ShapeDtypeStruct((NC, NPAD, D_HID), jnp.float32),
    mesh=_sc_mesh,
    compiler_params=_sc_params,
    scratch_types=[
        pltpu.VMEM_SHARED((NPAD, D_HID), jnp.float32),
        pltpu.VMEM_SHARED((NPAD, D_HID), jnp.float32),
        pltpu.VMEM((K, B), jnp.int32),
        pltpu.VMEM((K, B), jnp.int32),
        pltpu.VMEM((B, D_HID), jnp.float32),
        pltpu.VMEM((RPC, D_HID), jnp.float32),
        pltpu.SemaphoreType.DMA,
    ],
)


def _prep_body(x_ref, w1_ref, degp_ref, hs1_ref, dis_ref):
  xw = jnp.dot(x_ref[...], w1_ref[...], preferred_element_type=jnp.float32)
  deg = degp_ref[0] + degp_ref[1] + 1.0
  dis = lax.rsqrt(jnp.maximum(deg, 1.0))
  dis_ref[...] = dis
  hs1_ref[...] = xw * dis


def _mid_body(aggp_ref, hs1_ref, dis_ref, hs2_ref):
  a = aggp_ref[0] + aggp_ref[1] + hs1_ref[...]
  hs2_ref[...] = jax.nn.relu(a * dis_ref[...]) * dis_ref[...]


def _fin_body(aggp_ref, hs2_ref, dis_ref, w2_ref, out_ref):
  a = (aggp_ref[0] + aggp_ref[1] + hs2_ref[...]) * dis_ref[...]
  out_ref[...] = jnp.dot(a, w2_ref[...], preferred_element_type=jnp.float32)


def kernel(x, edge_index, W1, W2):
  f32 = jnp.float32
  x_pad = jnp.pad(x, ((0, NPAD - N), (0, 0)))
  src = edge_index[0]
  dst = edge_index[1]
  pad = EPAD - E
  src_p = jnp.concatenate([src, jnp.zeros((pad,), jnp.int32)])
  dst_p = jnp.concatenate([dst, jnp.full((pad,), N, jnp.int32)])
  srcs = src_p.reshape(NW, K, B)
  dsts = dst_p.reshape(NW, K, B)
  zeros_h = jnp.zeros((NPAD, D_HID), f32)
  ones_h = jnp.ones((B, D_HID), f32)

  degp = _deg_sc(dsts, zeros_h, ones_h)

  hs1, dis = pl.pallas_call(
      _prep_body,
      out_shape=(jax.ShapeDtypeStruct((NPAD, D_HID), f32),
                 jax.ShapeDtypeStruct((NPAD, D_HID), f32)),
  )(x_pad, W1, degp)

  agg1 = _agg_sc(hs1, srcs, dsts, zeros_h)

  hs2 = pl.pallas_call(
      _mid_body,
      out_shape=jax.ShapeDtypeStruct((NPAD, D_HID), f32),
  )(agg1, hs1, dis)

  agg2 = _agg_sc(hs2, srcs, dsts, zeros_h)

  out = pl.pallas_call(
      _fin_body,
      out_shape=jax.ShapeDtypeStruct((NPAD, D_OUT), f32),
  )(agg2, hs2, dis, W2)

  return out[:N]


# speedup vs baseline: 70.8013x; 1.3601x over previous
import jax
import jax.numpy as jnp
from jax import lax
from jax.experimental import pallas as pl
from jax.experimental.pallas import tpu as pltpu
from jax.experimental.pallas import tpu_sc as plsc

N = 10000
E = 320000
D_IN = 128
D_HID = 16
D_OUT = 128

NC = 2
NS = 16
NW = NC * NS
B = 128
GTOT = E // B
K = 79
KB = K * B
NPAD = 10112
RPC = NPAD // NS
DEPTH = 4

_sc_mesh = plsc.VectorSubcoreMesh(
    core_axis_name="c", subcore_axis_name="s", num_cores=NC, num_subcores=NS)
_sc_params = pltpu.CompilerParams(use_tc_tiling_on_sc=False)

_f32 = jnp.float32


def _worker_span(c, s):
  wid = s * NC + c
  gbase = jnp.minimum(wid * K, GTOT - K)
  j0 = wid * K - gbase
  return gbase, j0


def _zero_rows(buf, n):
  def st(i, carry):
    buf[i] = jnp.zeros((D_HID,), _f32)
    return carry
  lax.fori_loop(0, n, st, 0)


def _deg_body(ei_hbm, out_hbm, acc_sh, dst_1d, ones_v, zbuf, sem):
  c = lax.axis_index("c")
  s = lax.axis_index("s")
  gbase, j0 = _worker_span(c, s)
  rows = pl.ds(s * RPC, RPC)
  _zero_rows(zbuf, RPC)
  pltpu.sync_copy(zbuf, acc_sh.at[rows])

  def st1(i, carry):
    ones_v[i] = jnp.ones((D_HID,), _f32)
    return carry
  lax.fori_loop(0, B, st1, 0)
  pltpu.sync_copy(ei_hbm.at[1, pl.ds(gbase * B, KB)], dst_1d)
  plsc.subcore_barrier()

  def issue(j):
    pltpu.async_copy(ones_v, acc_sh.at[dst_1d.at[pl.ds(j * B, B)]], sem,
                     add=True)

  def wait_one():
    pltpu.make_async_copy(ones_v, acc_sh.at[dst_1d.at[pl.ds(0, B)]],
                          sem).wait()

  def prime(j, carry):
    issue(j0 + j)
    return carry
  lax.fori_loop(0, DEPTH, prime, 0)

  def step(j, carry):
    wait_one()
    issue(j)
    return carry
  lax.fori_loop(j0 + DEPTH, K, step, 0)

  def drain(j, carry):
    wait_one()
    return carry
  lax.fori_loop(0, DEPTH, drain, 0)

  plsc.subcore_barrier()
  pltpu.sync_copy(acc_sh.at[rows], zbuf)
  pltpu.sync_copy(zbuf, out_hbm.at[c, rows])


_deg_sc = pl.kernel(
    _deg_body,
    out_type=jax.ShapeDtypeStruct((NC, NPAD, D_HID), _f32),
    mesh=_sc_mesh,
    compiler_params=_sc_params,
    scratch_types=[
        pltpu.VMEM_SHARED((NPAD, D_HID), _f32),
        pltpu.VMEM((KB,), jnp.int32),
        pltpu.VMEM((B, D_HID), _f32),
        pltpu.VMEM((RPC, D_HID), _f32),
        pltpu.SemaphoreType.DMA,
    ],
)


def _edge_loop(j0, src_1d, dst_1d, hs_sh, acc_sh, msg0, msg1, sem0, sem1):
  def gather(j, buf, sem):
    pltpu.async_copy(hs_sh.at[src_1d.at[pl.ds(j * B, B)]], buf, sem)

  def gwait(buf, sem):
    pltpu.make_async_copy(hs_sh.at[src_1d.at[pl.ds(0, B)]], buf, sem).wait()

  def scat(j, buf):
    pltpu.sync_copy(buf, acc_sh.at[dst_1d.at[pl.ds(j * B, B)]], add=True)

  gather(j0, msg0, sem0)
  npairs = (K - 1 - j0) // 2

  def pair(p, carry):
    ja = j0 + 2 * p + 1
    gather(ja, msg1, sem1)
    gwait(msg0, sem0)
    scat(ja - 1, msg0)
    gather(ja + 1, msg0, sem0)
    gwait(msg1, sem1)
    scat(ja, msg1)
    return carry
  lax.fori_loop(0, npairs, pair, 0)
  gwait(msg0, sem0)
  scat(K - 1, msg0)


def _agg1_body(hs_hbm, ei_hbm, out_hbm,
               acc_sh, hs_sh, src_1d, dst_1d, msg0, msg1, zbuf, sem0, sem1):
  c = lax.axis_index("c")
  s = lax.axis_index("s")
  gbase, j0 = _worker_span(c, s)
  rows = pl.ds(s * RPC, RPC)
  pltpu.sync_copy(hs_hbm.at[rows], zbuf)
  pltpu.sync_copy(zbuf, hs_sh.at[rows])
  _zero_rows(zbuf, RPC)
  pltpu.sync_copy(zbuf, acc_sh.at[rows])
  pltpu.sync_copy(ei_hbm.at[0, pl.ds(gbase * B, KB)], src_1d)
  pltpu.sync_copy(ei_hbm.at[1, pl.ds(gbase * B, KB)], dst_1d)
  plsc.subcore_barrier()
  _edge_loop(j0, src_1d, dst_1d, hs_sh, acc_sh, msg0, msg1, sem0, sem1)
  plsc.subcore_barrier()
  pltpu.sync_copy(acc_sh.at[rows], zbuf)
  pltpu.sync_copy(zbuf, out_hbm.at[c, rows])


_agg1_sc = pl.kernel(
    _agg1_body,
    out_type=jax.ShapeDtypeStruct((NC, NPAD, D_HID), _f32),
    mesh=_sc_mesh,
    compiler_params=_sc_params,
    scratch_types=[
        pltpu.VMEM_SHARED((NPAD, D_HID), _f32),
        pltpu.VMEM_SHARED((NPAD, D_HID), _f32),
        pltpu.VMEM((KB,), jnp.int32),
        pltpu.VMEM((KB,), jnp.int32),
        pltpu.VMEM((B, D_HID), _f32),
        pltpu.VMEM((B, D_HID), _f32),
        pltpu.VMEM((RPC, D_HID), _f32),
        pltpu.SemaphoreType.DMA,
        pltpu.SemaphoreType.DMA,
    ],
)


def _agg2_body(aggp_hbm, hs1_hbm, dis_hbm, ei_hbm, out_hbm, hs2_hbm,
               acc_sh, hs_sh, src_1d, dst_1d, msg0, msg1,
               p0b, p1b, h1b, disb, zbuf, sem0, sem1):
  c = lax.axis_index("c")
  s = lax.axis_index("s")
  gbase, j0 = _worker_span(c, s)
  rows = pl.ds(s * RPC, RPC)
  pltpu.sync_copy(aggp_hbm.at[0, rows], p0b)
  pltpu.sync_copy(aggp_hbm.at[1, rows], p1b)
  pltpu.sync_copy(hs1_hbm.at[rows], h1b)
  pltpu.sync_copy(dis_hbm.at[rows], disb)

  def ew(i, carry):
    a = p0b[i] + p1b[i] + h1b[i]
    d = disb[i]
    zbuf[i] = jnp.maximum(a * d, 0.0) * d
    return carry
  lax.fori_loop(0, RPC, ew, 0)

  pltpu.sync_copy(zbuf, hs_sh.at[rows])

  @pl.when(c == 0)
  def _():
    pltpu.sync_copy(zbuf, hs2_hbm.at[rows])

  _zero_rows(zbuf, RPC)
  pltpu.sync_copy(zbuf, acc_sh.at[rows])
  pltpu.sync_copy(ei_hbm.at[0, pl.ds(gbase * B, KB)], src_1d)
  pltpu.sync_copy(ei_hbm.at[1, pl.ds(gbase * B, KB)], dst_1d)
  plsc.subcore_barrier()
  _edge_loop(j0, src_1d, dst_1d, hs_sh, acc_sh, msg0, msg1, sem0, sem1)
  plsc.subcore_barrier()
  pltpu.sync_copy(acc_sh.at[rows], zbuf)
  pltpu.sync_copy(zbuf, out_hbm.at[c, rows])


_agg2_sc = pl.kernel(
    _agg2_body,
    out_type=(jax.ShapeDtypeStruct((NC, NPAD, D_HID), _f32),
              jax.ShapeDtypeStruct((NPAD, D_HID), _f32)),
    mesh=_sc_mesh,
    compiler_params=_sc_params,
    scratch_types=[
        pltpu.VMEM_SHARED((NPAD, D_HID), _f32),
        pltpu.VMEM_SHARED((NPAD, D_HID), _f32),
        pltpu.VMEM((KB,), jnp.int32),
        pltpu.VMEM((KB,), jnp.int32),
        pltpu.VMEM((B, D_HID), _f32),
        pltpu.VMEM((B, D_HID), _f32),
        pltpu.VMEM((RPC, D_HID), _f32),
        pltpu.VMEM((RPC, D_HID), _f32),
        pltpu.VMEM((RPC, D_HID), _f32),
        pltpu.VMEM((RPC, D_HID), _f32),
        pltpu.VMEM((RPC, D_HID), _f32),
        pltpu.SemaphoreType.DMA,
        pltpu.SemaphoreType.DMA,
    ],
)


def _prep_body(x_ref, w1_ref, degp_ref, hs1_ref, dis_ref):
  xw = jnp.dot(x_ref[...], w1_ref[...], preferred_element_type=_f32)
  deg = degp_ref[0] + degp_ref[1] + 1.0
  dis = lax.rsqrt(deg)
  dis_ref[...] = dis
  hs1_ref[:N, :] = xw * dis[:N, :]


def _fin_body(aggp_ref, hs2_ref, dis_ref, w2_ref, out_ref):
  aggp = aggp_ref[...]
  a = (aggp[0, :N, :] + aggp[1, :N, :] + hs2_ref[:N, :]) * dis_ref[:N, :]
  out_ref[...] = jnp.dot(a, w2_ref[...], preferred_element_type=_f32)


def kernel(x, edge_index, W1, W2):
  degp = _deg_sc(edge_index)

  hs1, dis = pl.pallas_call(
      _prep_body,
      out_shape=(jax.ShapeDtypeStruct((NPAD, D_HID), _f32),
                 jax.ShapeDtypeStruct((NPAD, D_HID), _f32)),
  )(x, W1, degp)

  agg1 = _agg1_sc(hs1, edge_index)
  agg2, hs2 = _agg2_sc(agg1, hs1, dis, edge_index)

  out = pl.pallas_call(
      _fin_body,
      out_shape=jax.ShapeDtypeStruct((N, D_OUT), _f32),
  )(agg2, hs2, dis, W2)

  return out


# speedup vs baseline: 78.4866x; 1.1085x over previous
import jax
import jax.numpy as jnp
from jax import lax
from jax.experimental import pallas as pl
from jax.experimental.pallas import tpu as pltpu
from jax.experimental.pallas import tpu_sc as plsc

N = 10000
E = 320000
D_IN = 128
D_HID = 16
D_OUT = 128

NC = 2
NS = 16
NW = NC * NS
B = 128
GTOT = E // B
K = 79
KB = K * B
NPAD = 10240
RPC = NPAD // NS
DEPTH = 4

_sc_mesh = plsc.VectorSubcoreMesh(
    core_axis_name="c", subcore_axis_name="s", num_cores=NC, num_subcores=NS)
_sc_params = pltpu.CompilerParams(use_tc_tiling_on_sc=False)

_f32 = jnp.float32


def _worker_span(c, s):
  wid = s * NC + c
  gbase = jnp.minimum(wid * K, GTOT - K)
  j0 = wid * K - gbase
  return gbase, j0


def _fill_rows(buf, n, val):
  def st(i, carry):
    buf[i] = jnp.full((D_HID,), val, _f32)
    return carry
  lax.fori_loop(0, n, st, 0)


def _fill_flat(buf, n, val):
  def st(i, carry):
    buf[pl.ds(i * D_HID, D_HID)] = jnp.full((D_HID,), val, _f32)
    return carry
  lax.fori_loop(0, n // D_HID, st, 0)


def _deg_body(ei_hbm, out_hbm, acc_sh, dst_1d, ones_v, zbuf, sem0, sem1):
  c = lax.axis_index("c")
  s = lax.axis_index("s")
  gbase, j0 = _worker_span(c, s)
  rows = pl.ds(s * RPC, RPC)
  pltpu.async_copy(ei_hbm.at[1, pl.ds(gbase * B, KB)], dst_1d, sem1)
  _fill_flat(zbuf, RPC, 0.0)
  _fill_flat(ones_v, B, 1.0)
  pltpu.sync_copy(zbuf, acc_sh.at[rows])
  pltpu.make_async_copy(ei_hbm.at[1, pl.ds(0, KB)], dst_1d, sem1).wait()
  plsc.subcore_barrier()

  def issue(j):
    pltpu.async_copy(ones_v, acc_sh.at[dst_1d.at[pl.ds(j * B, B)]], sem0,
                     add=True)

  def wait_one(carry=0):
    pltpu.make_async_copy(ones_v, acc_sh.at[dst_1d.at[pl.ds(0, B)]],
                          sem0).wait()
    return carry

  def prime(j, carry):
    issue(j0 + j)
    return carry
  lax.fori_loop(0, DEPTH, prime, 0)

  def step(j, carry):
    wait_one()
    issue(j)
    return carry
  lax.fori_loop(j0 + DEPTH, K, step, 0)

  def drain(j, carry):
    return wait_one(carry)
  lax.fori_loop(0, DEPTH, drain, 0)

  plsc.subcore_barrier()
  pltpu.sync_copy(acc_sh.at[rows], zbuf)
  pltpu.sync_copy(zbuf, out_hbm.at[c, rows])


_deg_sc = pl.kernel(
    _deg_body,
    out_type=jax.ShapeDtypeStruct((NC, NPAD), _f32),
    mesh=_sc_mesh,
    compiler_params=_sc_params,
    scratch_types=[
        pltpu.VMEM_SHARED((NPAD,), _f32),
        pltpu.VMEM((KB,), jnp.int32),
        pltpu.VMEM((B,), _f32),
        pltpu.VMEM((RPC,), _f32),
        pltpu.SemaphoreType.DMA,
        pltpu.SemaphoreType.DMA,
    ],
)


def _edge_loop(j0, src_1d, dst_1d, hs_sh, acc_sh, m0, m1, s0, s1, t0, t1):
  def g(j, buf, sem):
    pltpu.async_copy(hs_sh.at[src_1d.at[pl.ds(j * B, B)]], buf, sem)

  def gw(buf, sem):
    pltpu.make_async_copy(hs_sh.at[src_1d.at[pl.ds(0, B)]], buf, sem).wait()

  def sc(j, buf, sem):
    pltpu.async_copy(buf, acc_sh.at[dst_1d.at[pl.ds(j * B, B)]], sem,
                     add=True)

  def sw(buf, sem):
    pltpu.make_async_copy(buf, acc_sh.at[dst_1d.at[pl.ds(0, B)]], sem).wait()

  g(j0, m0, s0)
  g(j0 + 1, m1, s1)
  npairs = (K - j0 - 3) // 2

  def pair(p, carry):
    a = j0 + 2 * p
    gw(m0, s0)
    sc(a, m0, t0)
    gw(m1, s1)
    sc(a + 1, m1, t1)
    sw(m0, t0)
    g(a + 2, m0, s0)
    sw(m1, t1)
    g(a + 3, m1, s1)
    return carry
  lax.fori_loop(0, npairs, pair, 0)

  gw(m0, s0)
  sc(K - 3, m0, t0)
  sw(m0, t0)
  g(K - 1, m0, s0)
  gw(m1, s1)
  sc(K - 2, m1, t1)
  gw(m0, s0)
  sc(K - 1, m0, t0)
  sw(m0, t0)
  sw(m1, t1)


def _agg1_body(hs_hbm, ei_hbm, out_hbm,
               acc_sh, hs_sh, src_1d, dst_1d, m0, m1, zbuf, s0, s1, t0, t1):
  c = lax.axis_index("c")
  s = lax.axis_index("s")
  gbase, j0 = _worker_span(c, s)
  rows = pl.ds(s * RPC, RPC)
  pltpu.async_copy(hs_hbm.at[rows], hs_sh.at[rows], s0)
  pltpu.async_copy(ei_hbm.at[0, pl.ds(gbase * B, KB)], src_1d, s1)
  pltpu.async_copy(ei_hbm.at[1, pl.ds(gbase * B, KB)], dst_1d, s1)
  _fill_rows(zbuf, RPC, 0.0)
  pltpu.sync_copy(zbuf, acc_sh.at[rows])
  pltpu.make_async_copy(hs_hbm.at[rows], hs_sh.at[rows], s0).wait()
  pltpu.make_async_copy(ei_hbm.at[0, pl.ds(0, KB)], src_1d, s1).wait()
  pltpu.make_async_copy(ei_hbm.at[0, pl.ds(0, KB)], dst_1d, s1).wait()
  plsc.subcore_barrier()
  _edge_loop(j0, src_1d, dst_1d, hs_sh, acc_sh, m0, m1, s0, s1, t0, t1)
  plsc.subcore_barrier()
  pltpu.sync_copy(acc_sh.at[rows], zbuf)
  pltpu.sync_copy(zbuf, out_hbm.at[c, rows])


_agg1_sc = pl.kernel(
    _agg1_body,
    out_type=jax.ShapeDtypeStruct((NC, NPAD, D_HID), _f32),
    mesh=_sc_mesh,
    compiler_params=_sc_params,
    scratch_types=[
        pltpu.VMEM_SHARED((NPAD, D_HID), _f32),
        pltpu.VMEM_SHARED((NPAD, D_HID), _f32),
        pltpu.VMEM((KB,), jnp.int32),
        pltpu.VMEM((KB,), jnp.int32),
        pltpu.VMEM((B, D_HID), _f32),
        pltpu.VMEM((B, D_HID), _f32),
        pltpu.VMEM((RPC, D_HID), _f32),
        pltpu.SemaphoreType.DMA,
        pltpu.SemaphoreType.DMA,
        pltpu.SemaphoreType.DMA,
        pltpu.SemaphoreType.DMA,
    ],
)


def _agg2_body(aggp_hbm, hs1_hbm, dis_hbm, ei_hbm, out_hbm, hs2_hbm,
               acc_sh, hs_sh, src_1d, dst_1d, m0, m1,
               p0b, p1b, h1b, disb, zbuf, s0, s1, t0, t1):
  c = lax.axis_index("c")
  s = lax.axis_index("s")
  gbase, j0 = _worker_span(c, s)
  rows = pl.ds(s * RPC, RPC)
  pltpu.async_copy(aggp_hbm.at[0, rows], p0b, s0)
  pltpu.async_copy(aggp_hbm.at[1, rows], p1b, s0)
  pltpu.async_copy(hs1_hbm.at[rows], h1b, s0)
  pltpu.async_copy(dis_hbm.at[rows], disb, s0)
  pltpu.async_copy(ei_hbm.at[0, pl.ds(gbase * B, KB)], src_1d, s1)
  pltpu.async_copy(ei_hbm.at[1, pl.ds(gbase * B, KB)], dst_1d, s1)
  pltpu.make_async_copy(aggp_hbm.at[0, rows], p0b, s0).wait()
  pltpu.make_async_copy(aggp_hbm.at[1, rows], p1b, s0).wait()
  pltpu.make_async_copy(hs1_hbm.at[rows], h1b, s0).wait()
  pltpu.make_async_copy(dis_hbm.at[rows], disb, s0).wait()

  def ew(i, carry):
    a = p0b[i] + p1b[i] + h1b[i]
    d = disb[i]
    zbuf[i] = jnp.maximum(a * d, 0.0) * d
    return carry
  lax.fori_loop(0, RPC, ew, 0)

  pltpu.sync_copy(zbuf, hs_sh.at[rows])

  @pl.when(c == 0)
  def _():
    pltpu.sync_copy(zbuf, hs2_hbm.at[rows])

  _fill_rows(zbuf, RPC, 0.0)
  pltpu.sync_copy(zbuf, acc_sh.at[rows])
  pltpu.make_async_copy(ei_hbm.at[0, pl.ds(0, KB)], src_1d, s1).wait()
  pltpu.make_async_copy(ei_hbm.at[0, pl.ds(0, KB)], dst_1d, s1).wait()
  plsc.subcore_barrier()
  _edge_loop(j0, src_1d, dst_1d, hs_sh, acc_sh, m0, m1, s0, s1, t0, t1)
  plsc.subcore_barrier()
  pltpu.sync_copy(acc_sh.at[rows], zbuf)
  pltpu.sync_copy(zbuf, out_hbm.at[c, rows])


_agg2_sc = pl.kernel(
    _agg2_body,
    out_type=(jax.ShapeDtypeStruct((NC, NPAD, D_HID), _f32),
              jax.ShapeDtypeStruct((NPAD, D_HID), _f32)),
    mesh=_sc_mesh,
    compiler_params=_sc_params,
    scratch_types=[
        pltpu.VMEM_SHARED((NPAD, D_HID), _f32),
        pltpu.VMEM_SHARED((NPAD, D_HID), _f32),
        pltpu.VMEM((KB,), jnp.int32),
        pltpu.VMEM((KB,), jnp.int32),
        pltpu.VMEM((B, D_HID), _f32),
        pltpu.VMEM((B, D_HID), _f32),
        pltpu.VMEM((RPC, D_HID), _f32),
        pltpu.VMEM((RPC, D_HID), _f32),
        pltpu.VMEM((RPC, D_HID), _f32),
        pltpu.VMEM((RPC, D_HID), _f32),
        pltpu.VMEM((RPC, D_HID), _f32),
        pltpu.SemaphoreType.DMA,
        pltpu.SemaphoreType.DMA,
        pltpu.SemaphoreType.DMA,
        pltpu.SemaphoreType.DMA,
    ],
)


def _prep_body(x_ref, w1_ref, degp_ref, hs1_ref, dis_ref):
  xw = jnp.dot(x_ref[...], w1_ref[...], preferred_element_type=_f32)
  degp = degp_ref[...]
  deg = degp[0] + degp[1] + 1.0
  dis = jnp.broadcast_to(lax.rsqrt(deg)[:, None], (NPAD, D_HID))
  dis_ref[...] = dis
  hs1_ref[:N, :] = xw * dis[:N, :]


def _fin_body(aggp_ref, hs2_ref, dis_ref, w2_ref, out_ref):
  aggp = aggp_ref[...]
  a = (aggp[0, :N, :] + aggp[1, :N, :] + hs2_ref[:N, :]) * dis_ref[:N, :]
  out_ref[...] = jnp.dot(a, w2_ref[...], preferred_element_type=_f32)


def kernel(x, edge_index, W1, W2):
  degp = _deg_sc(edge_index)

  hs1, dis = pl.pallas_call(
      _prep_body,
      out_shape=(jax.ShapeDtypeStruct((NPAD, D_HID), _f32),
                 jax.ShapeDtypeStruct((NPAD, D_HID), _f32)),
  )(x, W1, degp)

  agg1 = _agg1_sc(hs1, edge_index)
  agg2, hs2 = _agg2_sc(agg1, hs1, dis, edge_index)

  out = pl.pallas_call(
      _fin_body,
      out_shape=jax.ShapeDtypeStruct((N, D_OUT), _f32),
  )(agg2, hs2, dis, W2)

  return out


# speedup vs baseline: 89.7774x; 1.1439x over previous
import jax
import jax.numpy as jnp
from jax import lax
from jax.experimental import pallas as pl
from jax.experimental.pallas import tpu as pltpu
from jax.experimental.pallas import tpu_sc as plsc

N = 10000
E = 320000
D_IN = 128
D_HID = 16
D_OUT = 128

NC = 2
NS = 16
NW = NC * NS
B = 128
GTOT = E // B
K = 79
KB = K * B
NPAD = 10240
RPC = NPAD // NS
DEPTH = 4
PH = 8
NPH = 10

_sc_mesh = plsc.VectorSubcoreMesh(
    core_axis_name="c", subcore_axis_name="s", num_cores=NC, num_subcores=NS)
_sc_params = pltpu.CompilerParams(use_tc_tiling_on_sc=False)

_f32 = jnp.float32


def _worker_span(c, s):
  wid = s * NC + c
  gbase = jnp.minimum(wid * K, GTOT - K)
  j0 = wid * K - gbase
  return gbase, j0


def _fill_rows(buf, n, val):
  def st(i, carry):
    buf[i] = jnp.full((D_HID,), val, _f32)
    return carry
  lax.fori_loop(0, n, st, 0)


def _fill_flat(buf, n, val):
  def st(i, carry):
    buf[pl.ds(i * D_HID, D_HID)] = jnp.full((D_HID,), val, _f32)
    return carry
  lax.fori_loop(0, n // D_HID, st, 0)


def _deg_body(ei_hbm, out_hbm, acc_sh, dst_1d, ones_v, zbuf, sem0, sem1):
  c = lax.axis_index("c")
  s = lax.axis_index("s")
  gbase, j0 = _worker_span(c, s)
  rows = pl.ds(s * RPC, RPC)
  pltpu.async_copy(ei_hbm.at[1, pl.ds(gbase * B, KB)], dst_1d, sem1)
  _fill_flat(zbuf, RPC, 0.0)
  _fill_flat(ones_v, B, 1.0)
  pltpu.sync_copy(zbuf, acc_sh.at[rows])
  pltpu.make_async_copy(ei_hbm.at[1, pl.ds(0, KB)], dst_1d, sem1).wait()
  plsc.subcore_barrier()

  def issue(j):
    pltpu.async_copy(ones_v, acc_sh.at[dst_1d.at[pl.ds(j * B, B)]], sem0,
                     add=True)

  def wait_one():
    pltpu.make_async_copy(ones_v, acc_sh.at[dst_1d.at[pl.ds(0, B)]],
                          sem0).wait()

  def prime(j, carry):
    issue(j0 + j)
    return carry
  lax.fori_loop(0, DEPTH, prime, 0)

  def step(j, carry):
    wait_one()
    issue(j)
    return carry
  lax.fori_loop(j0 + DEPTH, K, step, 0)

  def drain(j, carry):
    wait_one()
    return carry
  lax.fori_loop(0, DEPTH, drain, 0)

  plsc.subcore_barrier()
  pltpu.sync_copy(acc_sh.at[rows], zbuf)
  pltpu.sync_copy(zbuf, out_hbm.at[c, rows])


_deg_sc = pl.kernel(
    _deg_body,
    out_type=jax.ShapeDtypeStruct((NC, NPAD), _f32),
    mesh=_sc_mesh,
    compiler_params=_sc_params,
    scratch_types=[
        pltpu.VMEM_SHARED((NPAD,), _f32),
        pltpu.VMEM((KB,), jnp.int32),
        pltpu.VMEM((B,), _f32),
        pltpu.VMEM((RPC,), _f32),
        pltpu.SemaphoreType.DMA,
        pltpu.SemaphoreType.DMA,
    ],
)


def _edge_loop(j0, src_1d, dst_1d, hs_sh, acc_sh, mbig, sg, st):
  def nb_of(p):
    start = j0 + p * PH
    return jnp.maximum(0, jnp.minimum(PH, K - start))

  def g_issue_all(p):
    start = j0 + p * PH
    off = (p % 2) * PH
    def gi(i, carry):
      pltpu.async_copy(hs_sh.at[src_1d.at[pl.ds((start + i) * B, B)]],
                       mbig.at[off + i], sg)
      return carry
    lax.fori_loop(0, nb_of(p), gi, 0)

  def g_drain(p):
    def gw(i, carry):
      pltpu.make_async_copy(hs_sh.at[src_1d.at[pl.ds(0, B)]], mbig.at[0],
                            sg).wait()
      return carry
    lax.fori_loop(0, nb_of(p), gw, 0)

  def s_issue_all(p):
    start = j0 + p * PH
    off = (p % 2) * PH
    def si(i, carry):
      pltpu.async_copy(mbig.at[off + i],
                       acc_sh.at[dst_1d.at[pl.ds((start + i) * B, B)]], st,
                       add=True)
      return carry
    lax.fori_loop(0, nb_of(p), si, 0)

  def s_drain(p):
    def sw(i, carry):
      pltpu.make_async_copy(mbig.at[0],
                            acc_sh.at[dst_1d.at[pl.ds(0, B)]], st).wait()
      return carry
    lax.fori_loop(0, nb_of(p), sw, 0)

  g_issue_all(0)
  for p in range(NPH):
    g_drain(p)
    s_issue_all(p)
    if p + 1 < NPH:
      g_issue_all(p + 1)
    s_drain(p)


def _agg1_body(hs_hbm, ei_hbm, out_hbm,
               acc_sh, hs_sh, src_1d, dst_1d, mbig, zbuf, s0, s1, sg, st):
  c = lax.axis_index("c")
  s = lax.axis_index("s")
  gbase, j0 = _worker_span(c, s)
  rows = pl.ds(s * RPC, RPC)
  pltpu.async_copy(hs_hbm.at[rows, pl.ds(0, D_HID)], hs_sh.at[rows], s0)
  pltpu.async_copy(ei_hbm.at[0, pl.ds(gbase * B, KB)], src_1d, s1)
  pltpu.async_copy(ei_hbm.at[1, pl.ds(gbase * B, KB)], dst_1d, s1)
  _fill_rows(zbuf, RPC, 0.0)
  pltpu.sync_copy(zbuf, acc_sh.at[rows])
  pltpu.make_async_copy(hs_hbm.at[rows, pl.ds(0, D_HID)], hs_sh.at[rows],
                        s0).wait()
  pltpu.make_async_copy(ei_hbm.at[0, pl.ds(0, KB)], src_1d, s1).wait()
  pltpu.make_async_copy(ei_hbm.at[0, pl.ds(0, KB)], dst_1d, s1).wait()
  plsc.subcore_barrier()
  _edge_loop(j0, src_1d, dst_1d, hs_sh, acc_sh, mbig, sg, st)
  plsc.subcore_barrier()
  pltpu.sync_copy(acc_sh.at[rows], zbuf)
  pltpu.sync_copy(zbuf, out_hbm.at[c, rows])


_agg1_sc = pl.kernel(
    _agg1_body,
    out_type=jax.ShapeDtypeStruct((NC, NPAD, D_HID), _f32),
    mesh=_sc_mesh,
    compiler_params=_sc_params,
    scratch_types=[
        pltpu.VMEM_SHARED((NPAD, D_HID), _f32),
        pltpu.VMEM_SHARED((NPAD, D_HID), _f32),
        pltpu.VMEM((KB,), jnp.int32),
        pltpu.VMEM((KB,), jnp.int32),
        pltpu.VMEM((2 * PH, B, D_HID), _f32),
        pltpu.VMEM((RPC, D_HID), _f32),
        pltpu.SemaphoreType.DMA,
        pltpu.SemaphoreType.DMA,
        pltpu.SemaphoreType.DMA,
        pltpu.SemaphoreType.DMA,
    ],
)


def _agg2_body(aggp_hbm, hs1_hbm, dis_hbm, ei_hbm, out_hbm, hs2_hbm,
               acc_sh, hs_sh, src_1d, dst_1d, mbig,
               p0b, p1b, h1b, disb1, zbuf, s0, s1, sg, st):
  c = lax.axis_index("c")
  s = lax.axis_index("s")
  gbase, j0 = _worker_span(c, s)
  rows = pl.ds(s * RPC, RPC)
  pltpu.async_copy(aggp_hbm.at[0, rows], p0b, s0)
  pltpu.async_copy(aggp_hbm.at[1, rows], p1b, s0)
  pltpu.async_copy(hs1_hbm.at[rows, pl.ds(0, D_HID)], h1b, s0)
  pltpu.async_copy(dis_hbm.at[rows], disb1.at[pl.ds(0, RPC)], s0)
  pltpu.async_copy(ei_hbm.at[0, pl.ds(gbase * B, KB)], src_1d, s1)
  pltpu.async_copy(ei_hbm.at[1, pl.ds(gbase * B, KB)], dst_1d, s1)
  pltpu.make_async_copy(aggp_hbm.at[0, rows], p0b, s0).wait()
  pltpu.make_async_copy(aggp_hbm.at[1, rows], p1b, s0).wait()
  pltpu.make_async_copy(hs1_hbm.at[rows, pl.ds(0, D_HID)], h1b, s0).wait()
  pltpu.make_async_copy(dis_hbm.at[rows], disb1.at[pl.ds(0, RPC)], s0).wait()

  def ew(i, carry):
    a = p0b[i] + p1b[i] + h1b[i]
    d = disb1[pl.ds(i, D_HID)][0]
    zbuf[i] = jnp.maximum(a * d, 0.0) * d
    return carry
  lax.fori_loop(0, RPC, ew, 0)

  pltpu.sync_copy(zbuf, hs_sh.at[rows])

  @pl.when(c == 0)
  def _():
    pltpu.sync_copy(zbuf, hs2_hbm.at[rows])

  _fill_rows(zbuf, RPC, 0.0)
  pltpu.sync_copy(zbuf, acc_sh.at[rows])
  pltpu.make_async_copy(ei_hbm.at[0, pl.ds(0, KB)], src_1d, s1).wait()
  pltpu.make_async_copy(ei_hbm.at[0, pl.ds(0, KB)], dst_1d, s1).wait()
  plsc.subcore_barrier()
  _edge_loop(j0, src_1d, dst_1d, hs_sh, acc_sh, mbig, sg, st)
  plsc.subcore_barrier()
  pltpu.sync_copy(acc_sh.at[rows], zbuf)
  pltpu.sync_copy(zbuf, out_hbm.at[c, rows])


_agg2_sc = pl.kernel(
    _agg2_body,
    out_type=(jax.ShapeDtypeStruct((NC, NPAD, D_HID), _f32),
              jax.ShapeDtypeStruct((NPAD, D_HID), _f32)),
    mesh=_sc_mesh,
    compiler_params=_sc_params,
    scratch_types=[
        pltpu.VMEM_SHARED((NPAD, D_HID), _f32),
        pltpu.VMEM_SHARED((NPAD, D_HID), _f32),
        pltpu.VMEM((KB,), jnp.int32),
        pltpu.VMEM((KB,), jnp.int32),
        pltpu.VMEM((2 * PH, B, D_HID), _f32),
        pltpu.VMEM((RPC, D_HID), _f32),
        pltpu.VMEM((RPC, D_HID), _f32),
        pltpu.VMEM((RPC, D_HID), _f32),
        pltpu.VMEM((RPC + D_HID,), _f32),
        pltpu.VMEM((RPC, D_HID), _f32),
        pltpu.SemaphoreType.DMA,
        pltpu.SemaphoreType.DMA,
        pltpu.SemaphoreType.DMA,
        pltpu.SemaphoreType.DMA,
    ],
)


def _prep_body(x_ref, w1_ref, degp_ref, hs1_ref, dis_ref):
  xw = jnp.dot(x_ref[...], w1_ref[...], preferred_element_type=_f32)
  degp = degp_ref[...]
  deg = degp[0] + degp[1] + 1.0
  dis = lax.rsqrt(deg)
  dis_ref[...] = dis
  hs1_ref[:N, :D_HID] = xw * dis[:N, None]


def _fin_body(aggp_ref, hs2_ref, dis_ref, w2_ref, out_ref):
  aggp = aggp_ref[...]
  dis = dis_ref[...]
  a = (aggp[0, :N, :] + aggp[1, :N, :] + hs2_ref[:N, :]) * dis[:N, None]
  out_ref[...] = jnp.dot(a, w2_ref[...], preferred_element_type=_f32)


def kernel(x, edge_index, W1, W2):
  degp = _deg_sc(edge_index)

  hs1, dis = pl.pallas_call(
      _prep_body,
      out_shape=(jax.ShapeDtypeStruct((NPAD, D_IN), _f32),
                 jax.ShapeDtypeStruct((NPAD,), _f32)),
  )(x, W1, degp)

  agg1 = _agg1_sc(hs1, edge_index)
  agg2, hs2 = _agg2_sc(agg1, hs1, dis, edge_index)

  out = pl.pallas_call(
      _fin_body,
      out_shape=jax.ShapeDtypeStruct((N, D_OUT), _f32),
  )(agg2, hs2, dis, W2)

  return out


# speedup vs baseline: 97.2779x; 1.0835x over previous
import jax
import jax.numpy as jnp
from jax import lax
from jax.experimental import pallas as pl
from jax.experimental.pallas import tpu as pltpu
from jax.experimental.pallas import tpu_sc as plsc

N = 10000
E = 320000
D_IN = 128
D_HID = 16
D_OUT = 128

NC = 2
NS = 16
NW = NC * NS
B = 128
GTOT = E // B
K = 79
KB = K * B
NPAD = 10240
RPC = NPAD // NS
DEPTH = 4
GD = 8

_sc_mesh = plsc.VectorSubcoreMesh(
    core_axis_name="c", subcore_axis_name="s", num_cores=NC, num_subcores=NS)
_sc_params = pltpu.CompilerParams(use_tc_tiling_on_sc=False)

_f32 = jnp.float32


def _worker_span(c, s):
  wid = s * NC + c
  gbase = jnp.minimum(wid * K, GTOT - K)
  j0 = wid * K - gbase
  return gbase, j0


def _fill_rows(buf, n, val):
  def st(i, carry):
    buf[i] = jnp.full((D_HID,), val, _f32)
    return carry
  lax.fori_loop(0, n, st, 0)


def _fill_flat(buf, n, val):
  def st(i, carry):
    buf[pl.ds(i * D_HID, D_HID)] = jnp.full((D_HID,), val, _f32)
    return carry
  lax.fori_loop(0, n // D_HID, st, 0)


def _deg_body(ei_hbm, out_hbm, acc_sh, dst_1d, ones_v, zbuf, sem0, sem1):
  c = lax.axis_index("c")
  s = lax.axis_index("s")
  gbase, j0 = _worker_span(c, s)
  rows = pl.ds(s * RPC, RPC)
  pltpu.async_copy(ei_hbm.at[1, pl.ds(gbase * B, KB)], dst_1d, sem1)
  _fill_flat(zbuf, RPC, 0.0)
  _fill_flat(ones_v, B, 1.0)
  pltpu.sync_copy(zbuf, acc_sh.at[rows])
  pltpu.make_async_copy(ei_hbm.at[1, pl.ds(0, KB)], dst_1d, sem1).wait()
  plsc.subcore_barrier()

  def issue(j):
    pltpu.async_copy(ones_v, acc_sh.at[dst_1d.at[pl.ds(j * B, B)]], sem0,
                     add=True)

  def wait_one():
    pltpu.make_async_copy(ones_v, acc_sh.at[dst_1d.at[pl.ds(0, B)]],
                          sem0).wait()

  def prime(j, carry):
    issue(j0 + j)
    return carry
  lax.fori_loop(0, DEPTH, prime, 0)

  def step(j, carry):
    wait_one()
    issue(j)
    return carry
  lax.fori_loop(j0 + DEPTH, K, step, 0)

  def drain(j, carry):
    wait_one()
    return carry
  lax.fori_loop(0, DEPTH, drain, 0)

  plsc.subcore_barrier()
  pltpu.sync_copy(acc_sh.at[rows], zbuf)
  pltpu.sync_copy(zbuf, out_hbm.at[c, rows])


_deg_sc = pl.kernel(
    _deg_body,
    out_type=jax.ShapeDtypeStruct((NC, NPAD), _f32),
    mesh=_sc_mesh,
    compiler_params=_sc_params,
    scratch_types=[
        pltpu.VMEM_SHARED((NPAD,), _f32),
        pltpu.VMEM((KB,), jnp.int32),
        pltpu.VMEM((B,), _f32),
        pltpu.VMEM((RPC,), _f32),
        pltpu.SemaphoreType.DMA,
        pltpu.SemaphoreType.DMA,
    ],
)


def _edge_loop(j0, src_1d, dst_1d, hs_sh, acc_sh, mbig, sg, st):
  def row(j):
    return lax.rem(j - j0, 2 * GD)

  def g_issue(j):
    pltpu.async_copy(hs_sh.at[src_1d.at[pl.ds(j * B, B)]], mbig.at[row(j)],
                     sg)

  def g_wait():
    pltpu.make_async_copy(hs_sh.at[src_1d.at[pl.ds(0, B)]], mbig.at[0],
                          sg).wait()

  def s_issue(j):
    pltpu.async_copy(mbig.at[row(j)],
                     acc_sh.at[dst_1d.at[pl.ds(j * B, B)]], st, add=True)

  def s_wait():
    pltpu.make_async_copy(mbig.at[0], acc_sh.at[dst_1d.at[pl.ds(0, B)]],
                          st).wait()

  def pro(i, carry):
    g_issue(j0 + i)
    return carry
  lax.fori_loop(0, GD, pro, 0)

  def l1(j, carry):
    g_wait()
    s_issue(j)
    g_issue(j + GD)
    return carry
  lax.fori_loop(j0, j0 + GD, l1, 0)

  def l2(j, carry):
    g_wait()
    s_issue(j)
    s_wait()
    g_issue(j + GD)
    return carry
  lax.fori_loop(j0 + GD, K - GD, l2, 0)

  def l3(j, carry):
    g_wait()
    s_issue(j)
    s_wait()
    return carry
  lax.fori_loop(K - GD, K, l3, 0)

  def l4(i, carry):
    s_wait()
    return carry
  lax.fori_loop(0, GD, l4, 0)


def _agg1_body(hs_hbm, ei_hbm, out_hbm,
               acc_sh, hs_sh, src_1d, dst_1d, mbig, zbuf, s0, s1, sg, st):
  c = lax.axis_index("c")
  s = lax.axis_index("s")
  gbase, j0 = _worker_span(c, s)
  rows = pl.ds(s * RPC, RPC)
  pltpu.async_copy(hs_hbm.at[rows, pl.ds(0, D_HID)], hs_sh.at[rows], s0)
  pltpu.async_copy(ei_hbm.at[0, pl.ds(gbase * B, KB)], src_1d, s1)
  pltpu.async_copy(ei_hbm.at[1, pl.ds(gbase * B, KB)], dst_1d, s1)
  _fill_rows(zbuf, RPC, 0.0)
  pltpu.sync_copy(zbuf, acc_sh.at[rows])
  pltpu.make_async_copy(hs_hbm.at[rows, pl.ds(0, D_HID)], hs_sh.at[rows],
                        s0).wait()
  pltpu.make_async_copy(ei_hbm.at[0, pl.ds(0, KB)], src_1d, s1).wait()
  pltpu.make_async_copy(ei_hbm.at[0, pl.ds(0, KB)], dst_1d, s1).wait()
  plsc.subcore_barrier()
  _edge_loop(j0, src_1d, dst_1d, hs_sh, acc_sh, mbig, sg, st)
  plsc.subcore_barrier()
  pltpu.sync_copy(acc_sh.at[rows], zbuf)
  pltpu.sync_copy(zbuf, out_hbm.at[c, rows])


_agg1_sc = pl.kernel(
    _agg1_body,
    out_type=jax.ShapeDtypeStruct((NC, NPAD, D_HID), _f32),
    mesh=_sc_mesh,
    compiler_params=_sc_params,
    scratch_types=[
        pltpu.VMEM_SHARED((NPAD, D_HID), _f32),
        pltpu.VMEM_SHARED((NPAD, D_HID), _f32),
        pltpu.VMEM((KB,), jnp.int32),
        pltpu.VMEM((KB,), jnp.int32),
        pltpu.VMEM((2 * GD, B, D_HID), _f32),
        pltpu.VMEM((RPC, D_HID), _f32),
        pltpu.SemaphoreType.DMA,
        pltpu.SemaphoreType.DMA,
        pltpu.SemaphoreType.DMA,
        pltpu.SemaphoreType.DMA,
    ],
)


def _agg2_body(aggp_hbm, hs1_hbm, dis_hbm, ei_hbm, out_hbm, hs2_hbm,
               acc_sh, hs_sh, src_1d, dst_1d, mbig,
               p0b, p1b, h1b, disb1, zbuf, zbufz, s0, s1, sg, st):
  c = lax.axis_index("c")
  s = lax.axis_index("s")
  gbase, j0 = _worker_span(c, s)
  rows = pl.ds(s * RPC, RPC)
  pltpu.async_copy(aggp_hbm.at[0, rows], p0b, s0)
  pltpu.async_copy(aggp_hbm.at[1, rows], p1b, s0)
  pltpu.async_copy(hs1_hbm.at[rows, pl.ds(0, D_HID)], h1b, s0)
  pltpu.async_copy(dis_hbm.at[rows], disb1.at[pl.ds(0, RPC)], s0)
  pltpu.async_copy(ei_hbm.at[0, pl.ds(gbase * B, KB)], src_1d, s1)
  pltpu.async_copy(ei_hbm.at[1, pl.ds(gbase * B, KB)], dst_1d, s1)
  _fill_rows(zbufz, RPC, 0.0)
  pltpu.sync_copy(zbufz, acc_sh.at[rows])
  pltpu.make_async_copy(aggp_hbm.at[0, rows], p0b, s0).wait()
  pltpu.make_async_copy(aggp_hbm.at[1, rows], p1b, s0).wait()
  pltpu.make_async_copy(hs1_hbm.at[rows, pl.ds(0, D_HID)], h1b, s0).wait()
  pltpu.make_async_copy(dis_hbm.at[rows], disb1.at[pl.ds(0, RPC)], s0).wait()

  def ew(i, carry):
    a = p0b[i] + p1b[i] + h1b[i]
    d = disb1[pl.ds(i, D_HID)][0]
    zbuf[i] = jnp.maximum(a * d, 0.0) * d
    return carry
  lax.fori_loop(0, RPC, ew, 0)

  pltpu.async_copy(zbuf, hs_sh.at[rows], sg)

  @pl.when(c == 0)
  def _():
    pltpu.async_copy(zbuf, hs2_hbm.at[rows], st)

  pltpu.make_async_copy(ei_hbm.at[0, pl.ds(0, KB)], src_1d, s1).wait()
  pltpu.make_async_copy(ei_hbm.at[0, pl.ds(0, KB)], dst_1d, s1).wait()
  pltpu.make_async_copy(zbuf, hs_sh.at[rows], sg).wait()

  @pl.when(c == 0)
  def _():
    pltpu.make_async_copy(zbuf, hs2_hbm.at[rows], st).wait()

  plsc.subcore_barrier()
  _edge_loop(j0, src_1d, dst_1d, hs_sh, acc_sh, mbig, sg, st)
  plsc.subcore_barrier()
  pltpu.sync_copy(acc_sh.at[rows], zbuf)
  pltpu.sync_copy(zbuf, out_hbm.at[c, rows])


_agg2_sc = pl.kernel(
    _agg2_body,
    out_type=(jax.ShapeDtypeStruct((NC, NPAD, D_HID), _f32),
              jax.ShapeDtypeStruct((NPAD, D_HID), _f32)),
    mesh=_sc_mesh,
    compiler_params=_sc_params,
    scratch_types=[
        pltpu.VMEM_SHARED((NPAD, D_HID), _f32),
        pltpu.VMEM_SHARED((NPAD, D_HID), _f32),
        pltpu.VMEM((KB,), jnp.int32),
        pltpu.VMEM((KB,), jnp.int32),
        pltpu.VMEM((2 * GD, B, D_HID), _f32),
        pltpu.VMEM((RPC, D_HID), _f32),
        pltpu.VMEM((RPC, D_HID), _f32),
        pltpu.VMEM((RPC, D_HID), _f32),
        pltpu.VMEM((RPC + D_HID,), _f32),
        pltpu.VMEM((RPC, D_HID), _f32),
        pltpu.VMEM((RPC, D_HID), _f32),
        pltpu.SemaphoreType.DMA,
        pltpu.SemaphoreType.DMA,
        pltpu.SemaphoreType.DMA,
        pltpu.SemaphoreType.DMA,
    ],
)


def _mm_body(x_ref, w1_ref, xw_ref):
  xw_ref[...] = jnp.dot(x_ref[...], w1_ref[...], preferred_element_type=_f32)


def _scale_body(xw_ref, degp_ref, hs1_ref, dis_ref):
  degp = degp_ref[...]
  deg = degp[0] + degp[1] + 1.0
  dis = lax.rsqrt(deg)
  dis_ref[...] = dis
  hs1_ref[:N, :D_HID] = xw_ref[...] * dis[:N, None]


def _fin_body(aggp_ref, hs2_ref, dis_ref, w2_ref, out_ref):
  aggp = aggp_ref[...]
  dis = dis_ref[...]
  a = (aggp[0, :N, :] + aggp[1, :N, :] + hs2_ref[:N, :]) * dis[:N, None]
  out_ref[...] = jnp.dot(a, w2_ref[...], preferred_element_type=_f32)


def kernel(x, edge_index, W1, W2):
  xw = pl.pallas_call(
      _mm_body,
      out_shape=jax.ShapeDtypeStruct((N, D_HID), _f32),
  )(x, W1)

  degp = _deg_sc(edge_index)

  hs1, dis = pl.pallas_call(
      _scale_body,
      out_shape=(jax.ShapeDtypeStruct((NPAD, D_IN), _f32),
                 jax.ShapeDtypeStruct((NPAD,), _f32)),
  )(xw, degp)

  agg1 = _agg1_sc(hs1, edge_index)
  agg2, hs2 = _agg2_sc(agg1, hs1, dis, edge_index)

  out = pl.pallas_call(
      _fin_body,
      out_shape=jax.ShapeDtypeStruct((N, D_OUT), _f32),
  )(agg2, hs2, dis, W2)

  return out
